# baseline (speedup 1.0000x reference)
import functools

import jax
import jax.numpy as jnp
from jax import lax
from jax.experimental import pallas as pl
from jax.experimental.pallas import tpu as pltpu
from jax.experimental.pallas import tpu_sc as plsc

BATCH = 16384
EMBED_DIM = 32
HIDDEN_DIM = 64
CHUNK = 128
PACK = 128 // EMBED_DIM

_info = plsc.get_sparse_core_info()
NC, NS = _info.num_cores, _info.num_subcores
NW = NC * NS
B_PER_W = BATCH // NW
NCH = B_PER_W // CHUNK


NROWS = 1000000
FULL_BLOCKS = NROWS // 128
NSLICE = FULL_BLOCKS * 32
NTAILR = NROWS - NSLICE * PACK
_EXTRA = FULL_BLOCKS - 244 * NW


def _sc_transpose(tt_u, tt_m):
    mesh = plsc.VectorSubcoreMesh(core_axis_name="c", subcore_axis_name="s")
    out_t = jax.ShapeDtypeStruct((NSLICE, 128), jnp.float32)

    @functools.partial(
        pl.kernel,
        mesh=mesh,
        out_type=[out_t, out_t],
        compiler_params=pltpu.CompilerParams(needs_layout_passes=False),
        scratch_types=[
            pltpu.VMEM((32, 128), jnp.float32),
            pltpu.VMEM((32, 128), jnp.float32),
            pltpu.VMEM((32, 128), jnp.float32),
            pltpu.VMEM((32, 128), jnp.float32),
            pltpu.SemaphoreType.DMA,
            pltpu.SemaphoreType.DMA,
            pltpu.SemaphoreType.DMA,
            pltpu.SemaphoreType.DMA,
        ],
    )
    def k(tu, tm, u_out, m_out, vin0, vin1, vout0, vout1,
          is0, is1, os0, os1):
        wid = lax.axis_index("s") * NC + lax.axis_index("c")
        start = wid * 244 + 2 * jnp.minimum(wid, 2)
        cnt = jnp.where(wid < 2, 246, 244)
        iota = lax.iota(jnp.int32, 16)
        rowvecs = [(h % 2) * 16 + iota for h in range(8)]
        vins = [vin0, vin1]
        vouts = [vout0, vout1]
        isems = [is0, is1]
        osems = [os0, os1]

        def extract(vin, vout):
            for r in range(32):
                for h in range(8):
                    colv = jnp.full((16,), 4 * r + h // 2, jnp.int32)
                    vals = plsc.load_gather(vin, [rowvecs[h], colv])
                    vout[r, pl.ds(16 * h, 16)] = vals

        def run_table(tt, out):
            pltpu.async_copy(tt.at[:, pl.ds(start * 128, 128)], vins[0],
                             isems[0])

            def body(k2, _):
                for par in range(2):
                    kk = 2 * k2 + par
                    b = start + kk
                    pltpu.make_async_copy(
                        tt.at[:, pl.ds(0, 128)], vins[par], isems[par]).wait()
                    @pl.when(kk + 1 < cnt)
                    def _():
                        pltpu.async_copy(
                            tt.at[:, pl.ds((b + 1) * 128, 128)],
                            vins[1 - par], isems[1 - par])
                    @pl.when(kk >= 2)
                    def _():
                        pltpu.make_async_copy(
                            vouts[par], out.at[pl.ds(0, 32)],
                            osems[par]).wait()
                    extract(vins[par], vouts[par])
                    pltpu.async_copy(vouts[par],
                                     out.at[pl.ds(b * 32, 32)], osems[par])
                return 0

            lax.fori_loop(0, cnt // 2, body, 0)
            for par in range(2):
                pltpu.make_async_copy(vouts[par], out.at[pl.ds(0, 32)],
                                      osems[par]).wait()

        run_table(tu, u_out)
        run_table(tm, m_out)

    return k(tt_u, tt_m)


def _sc_gather(ut, mt, uidx, m1idx, m2idx):
    mesh = plsc.VectorSubcoreMesh(core_axis_name="c", subcore_axis_name="s")
    out_t = jax.ShapeDtypeStruct((BATCH, 128), jnp.float32)

    @functools.partial(
        pl.kernel,
        mesh=mesh,
        out_type=[out_t, out_t, out_t],
        scratch_types=[
            pltpu.VMEM((NCH, CHUNK), jnp.int32),
            pltpu.VMEM((NCH, CHUNK), jnp.int32),
            pltpu.VMEM((NCH, CHUNK), jnp.int32),
        ] + [pltpu.VMEM((CHUNK, 128), jnp.float32) for _ in range(6)]
          + [pltpu.SemaphoreType.DMA for _ in range(6)],
    )
    def k(ut_hbm, mt_hbm, ui_hbm, m1i_hbm, m2i_hbm,
          u_out, m1_out, m2_out,
          ui_v, m1i_v, m2i_v, b0, b1_, b2_, b3, b4, b5,
          s0, s1, s2, s3, s4, s5):
        wid = lax.axis_index("s") * NC + lax.axis_index("c")
        base = wid * B_PER_W
        pltpu.sync_copy(ui_hbm.at[wid], ui_v)
        pltpu.sync_copy(m1i_hbm.at[wid], m1i_v)
        pltpu.sync_copy(m2i_hbm.at[wid], m2i_v)
        tabs = [(ut_hbm, ui_v, u_out), (mt_hbm, m1i_v, m1_out),
                (mt_hbm, m2i_v, m2_out)]
        bufs = [b0, b1_, b2_, b3, b4, b5]
        sems = [s0, s1, s2, s3, s4, s5]
        gd = {}
        for t in range(3):
            for s in range(2):
                tbl, idxv, _ = tabs[t]
                gd[(t, s)] = pltpu.async_copy(
                    tbl.at[idxv.at[s]], bufs[2 * t + s], sems[2 * t + s])
        od = {}
        for rnd in range(NCH // 2):
            for t in range(3):
                for s in range(2):
                    ch = 2 * rnd + s
                    tbl, idxv, out = tabs[t]
                    gd[(t, s)].wait()
                    od[(t, s)] = pltpu.async_copy(
                        bufs[2 * t + s],
                        out.at[pl.ds(base + ch * CHUNK, CHUNK)],
                        sems[2 * t + s])
            if rnd + 1 < NCH // 2:
                for t in range(3):
                    for s in range(2):
                        tbl, idxv, _ = tabs[t]
                        od[(t, s)].wait()
                        gd[(t, s)] = pltpu.async_copy(
                            tbl.at[idxv.at[2 * (rnd + 1) + s]],
                            bufs[2 * t + s], sems[2 * t + s])
        for t in range(3):
            for s in range(2):
                od[(t, s)].wait()

    return k(ut, mt, uidx, m1idx, m2idx)


_BLK = 2048


def _mlp_body(u_ref, m1_ref, m2_ref, us_ref, m1s_ref, m2s_ref,
              tu_ref, tm1_ref, tm2_ref, utail_ref, mtail_ref,
              w1u_ref, w1m_ref, b1_ref, w2_ref, out_ref):
    iota16 = lax.broadcasted_iota(jnp.int32, (1, NTAILR // PACK), 1)

    def fix(x4, tidx, tail):
        oh = (tidx == iota16).astype(jnp.float32)
        tg = jnp.dot(oh, tail, preferred_element_type=jnp.float32)
        return jnp.where(tidx >= 0, tg, x4)

    def pick(x4, sel):
        r = jnp.where(sel == 0, x4[:, 0 * EMBED_DIM:1 * EMBED_DIM], 0.0)
        for kk in range(1, PACK):
            r = r + jnp.where(sel == kk,
                              x4[:, kk * EMBED_DIM:(kk + 1) * EMBED_DIM], 0.0)
        return r

    u = pick(fix(u_ref[...], tu_ref[...], utail_ref[...]), us_ref[...])
    m1 = pick(fix(m1_ref[...], tm1_ref[...], mtail_ref[...]), m1s_ref[...])
    m2 = pick(fix(m2_ref[...], tm2_ref[...], mtail_ref[...]), m2s_ref[...])
    U = jnp.dot(u, w1u_ref[...], preferred_element_type=jnp.float32)
    M1 = jnp.dot(m1, w1m_ref[...], preferred_element_type=jnp.float32)
    M2 = jnp.dot(m2, w1m_ref[...], preferred_element_type=jnp.float32)
    b1r = b1_ref[...]
    h1 = jnp.maximum(U + M1 + b1r, 0.0)
    h2 = jnp.maximum(U + M2 + b1r, 0.0)
    out_ref[...] = jnp.sum((h1 - h2) * w2_ref[...], axis=1, keepdims=True)


def _tc_mlp(u4, m14, m24, usel, m1sel, m2sel,
            utidx, m1tidx, m2tidx, utail, mtail, W1, b1, W2):
    w1u = W1[:EMBED_DIM]
    w1m = W1[EMBED_DIM:]
    b1r = b1.reshape(1, HIDDEN_DIM)
    w2r = W2.reshape(1, HIDDEN_DIM)
    grid = (BATCH // _BLK,)
    ntail = NTAILR // PACK
    return pl.pallas_call(
        _mlp_body,
        grid=grid,
        in_specs=[
            pl.BlockSpec((_BLK, 128), lambda i: (i, 0)),
            pl.BlockSpec((_BLK, 128), lambda i: (i, 0)),
            pl.BlockSpec((_BLK, 128), lambda i: (i, 0)),
            pl.BlockSpec((_BLK, 1), lambda i: (i, 0)),
            pl.BlockSpec((_BLK, 1), lambda i: (i, 0)),
            pl.BlockSpec((_BLK, 1), lambda i: (i, 0)),
            pl.BlockSpec((_BLK, 1), lambda i: (i, 0)),
            pl.BlockSpec((_BLK, 1), lambda i: (i, 0)),
            pl.BlockSpec((_BLK, 1), lambda i: (i, 0)),
            pl.BlockSpec((ntail, 128), lambda i: (0, 0)),
            pl.BlockSpec((ntail, 128), lambda i: (0, 0)),
            pl.BlockSpec((EMBED_DIM, HIDDEN_DIM), lambda i: (0, 0)),
            pl.BlockSpec((EMBED_DIM, HIDDEN_DIM), lambda i: (0, 0)),
            pl.BlockSpec((1, HIDDEN_DIM), lambda i: (0, 0)),
            pl.BlockSpec((1, HIDDEN_DIM), lambda i: (0, 0)),
        ],
        out_specs=pl.BlockSpec((_BLK, 1), lambda i: (i, 0)),
        out_shape=jax.ShapeDtypeStruct((BATCH, 1), jnp.float32),
    )(u4, m14, m24, usel, m1sel, m2sel,
      utidx, m1tidx, m2tidx, utail, mtail, w1u, w1m, b1r, w2r)


def kernel(user_ids, movie_ids_1, movie_ids_2, user_table, movie_table,
           W1, b1, W2, b2):
    uid = user_ids.astype(jnp.int32)
    m1id = movie_ids_1.astype(jnp.int32)
    m2id = movie_ids_2.astype(jnp.int32)
    uq, m1q, m2q = uid // PACK, m1id // PACK, m2id // PACK
    uidx = jnp.minimum(uq, NSLICE - 1).reshape(NW, NCH, CHUNK)
    m1idx = jnp.minimum(m1q, NSLICE - 1).reshape(NW, NCH, CHUNK)
    m2idx = jnp.minimum(m2q, NSLICE - 1).reshape(NW, NCH, CHUNK)
    ut4t, mt4t = _sc_transpose(user_table.T, movie_table.T)
    u4, m14, m24 = _sc_gather(ut4t, mt4t, uidx, m1idx, m2idx)

    ntail = NTAILR // PACK
    utail = user_table[NSLICE * PACK:].reshape(ntail, 128)
    mtail = movie_table[NSLICE * PACK:].reshape(ntail, 128)
    return _tc_mlp(u4, m14, m24,
                   (uid % PACK).reshape(BATCH, 1),
                   (m1id % PACK).reshape(BATCH, 1),
                   (m2id % PACK).reshape(BATCH, 1),
                   (uq - NSLICE).reshape(BATCH, 1),
                   (m1q - NSLICE).reshape(BATCH, 1),
                   (m2q - NSLICE).reshape(BATCH, 1),
                   utail, mtail, W1, b1, W2)


# speedup vs baseline: 2.8748x; 2.8748x over previous
import functools

import jax
import jax.numpy as jnp
from jax import lax
from jax.experimental import pallas as pl
from jax.experimental.pallas import tpu as pltpu
from jax.experimental.pallas import tpu_sc as plsc

BATCH = 16384
EMBED_DIM = 32
HIDDEN_DIM = 64
CHUNK = 128
PACK = 128 // EMBED_DIM

_info = plsc.get_sparse_core_info()
NC, NS = _info.num_cores, _info.num_subcores
NW = NC * NS
B_PER_W = BATCH // NW
NCH = B_PER_W // CHUNK

NROWS = 1000000
QSHIFT = 18
NSLICE = 1 << QSHIFT
QMASK = NSLICE - 1

RB = 2048
RGRID = NSLICE // RB
MAXB = (NROWS - 1) // RB


def _repack_body(u0, u1, u2, u3, m0, m1, m2, m3, uo_ref, mo_ref):
    for p, r in enumerate((u0, u1, u2, u3)):
        uo_ref[:, 32 * p:32 * (p + 1)] = r[...].T
    for p, r in enumerate((m0, m1, m2, m3)):
        mo_ref[:, 32 * p:32 * (p + 1)] = r[...].T


def _tc_repack(tt_u, tt_m):
    out_t = jax.ShapeDtypeStruct((NSLICE, 128), jnp.float32)

    def spec(p):
        return pl.BlockSpec(
            (32, RB), lambda j, p=p: (0, jnp.minimum(128 * p + j, MAXB)))

    return pl.pallas_call(
        _repack_body,
        grid=(RGRID,),
        in_specs=[spec(p) for p in range(4)] * 2,
        out_specs=[pl.BlockSpec((RB, 128), lambda j: (j, 0))] * 2,
        out_shape=[out_t, out_t],
    )(*([tt_u] * 4 + [tt_m] * 4))


def _sc_gather(ut, mt, uidx, m1idx, m2idx):
    mesh = plsc.VectorSubcoreMesh(core_axis_name="c", subcore_axis_name="s")
    out_t = jax.ShapeDtypeStruct((BATCH, 128), jnp.float32)

    @functools.partial(
        pl.kernel,
        mesh=mesh,
        out_type=[out_t, out_t, out_t],
        compiler_params=pltpu.CompilerParams(use_tc_tiling_on_sc=True),
        scratch_types=[
            pltpu.VMEM((NCH, CHUNK), jnp.int32),
            pltpu.VMEM((NCH, CHUNK), jnp.int32),
            pltpu.VMEM((NCH, CHUNK), jnp.int32),
        ] + [pltpu.VMEM((CHUNK, 128), jnp.float32) for _ in range(6)]
          + [pltpu.SemaphoreType.DMA for _ in range(6)],
    )
    def k(ut_hbm, mt_hbm, ui_hbm, m1i_hbm, m2i_hbm,
          u_out, m1_out, m2_out,
          ui_v, m1i_v, m2i_v, b0, b1_, b2_, b3, b4, b5,
          s0, s1, s2, s3, s4, s5):
        wid = lax.axis_index("s") * NC + lax.axis_index("c")
        base = wid * B_PER_W
        pltpu.sync_copy(ui_hbm.at[wid], ui_v)
        pltpu.sync_copy(m1i_hbm.at[wid], m1i_v)
        pltpu.sync_copy(m2i_hbm.at[wid], m2i_v)
        tabs = [(ut_hbm, ui_v, u_out), (mt_hbm, m1i_v, m1_out),
                (mt_hbm, m2i_v, m2_out)]
        bufs = [b0, b1_, b2_, b3, b4, b5]
        sems = [s0, s1, s2, s3, s4, s5]
        gd = {}
        for t in range(3):
            for s in range(2):
                tbl, idxv, _ = tabs[t]
                gd[(t, s)] = pltpu.async_copy(
                    tbl.at[idxv.at[s]], bufs[2 * t + s], sems[2 * t + s])
        od = {}
        for rnd in range(NCH // 2):
            for t in range(3):
                for s in range(2):
                    ch = 2 * rnd + s
                    tbl, idxv, out = tabs[t]
                    gd[(t, s)].wait()
                    od[(t, s)] = pltpu.async_copy(
                        bufs[2 * t + s],
                        out.at[pl.ds(base + ch * CHUNK, CHUNK)],
                        sems[2 * t + s])
            if rnd + 1 < NCH // 2:
                for t in range(3):
                    for s in range(2):
                        tbl, idxv, _ = tabs[t]
                        od[(t, s)].wait()
                        gd[(t, s)] = pltpu.async_copy(
                            tbl.at[idxv.at[2 * (rnd + 1) + s]],
                            bufs[2 * t + s], sems[2 * t + s])
        for t in range(3):
            for s in range(2):
                od[(t, s)].wait()

    return k(ut, mt, uidx, m1idx, m2idx)


_BLK = 2048


def _mlp_body(u_ref, m1_ref, m2_ref, us_ref, m1s_ref, m2s_ref,
              w1u_ref, w1m_ref, b1_ref, w2_ref, out_ref):
    def pick(x4, sel):
        r = jnp.where(sel == 0, x4[:, 0 * EMBED_DIM:1 * EMBED_DIM], 0.0)
        for kk in range(1, PACK):
            r = r + jnp.where(sel == kk,
                              x4[:, kk * EMBED_DIM:(kk + 1) * EMBED_DIM], 0.0)
        return r

    u = pick(u_ref[...], us_ref[...])
    m1 = pick(m1_ref[...], m1s_ref[...])
    m2 = pick(m2_ref[...], m2s_ref[...])
    U = jnp.dot(u, w1u_ref[...], preferred_element_type=jnp.float32)
    M1 = jnp.dot(m1, w1m_ref[...], preferred_element_type=jnp.float32)
    M2 = jnp.dot(m2, w1m_ref[...], preferred_element_type=jnp.float32)
    b1r = b1_ref[...]
    h1 = jnp.maximum(U + M1 + b1r, 0.0)
    h2 = jnp.maximum(U + M2 + b1r, 0.0)
    out_ref[...] = jnp.sum((h1 - h2) * w2_ref[...], axis=1, keepdims=True)


def _tc_mlp(u4, m14, m24, usel, m1sel, m2sel, W1, b1, W2):
    w1u = W1[:EMBED_DIM]
    w1m = W1[EMBED_DIM:]
    b1r = b1.reshape(1, HIDDEN_DIM)
    w2r = W2.reshape(1, HIDDEN_DIM)
    grid = (BATCH // _BLK,)
    return pl.pallas_call(
        _mlp_body,
        grid=grid,
        in_specs=[
            pl.BlockSpec((_BLK, 128), lambda i: (i, 0)),
            pl.BlockSpec((_BLK, 128), lambda i: (i, 0)),
            pl.BlockSpec((_BLK, 128), lambda i: (i, 0)),
            pl.BlockSpec((_BLK, 1), lambda i: (i, 0)),
            pl.BlockSpec((_BLK, 1), lambda i: (i, 0)),
            pl.BlockSpec((_BLK, 1), lambda i: (i, 0)),
            pl.BlockSpec((EMBED_DIM, HIDDEN_DIM), lambda i: (0, 0)),
            pl.BlockSpec((EMBED_DIM, HIDDEN_DIM), lambda i: (0, 0)),
            pl.BlockSpec((1, HIDDEN_DIM), lambda i: (0, 0)),
            pl.BlockSpec((1, HIDDEN_DIM), lambda i: (0, 0)),
        ],
        out_specs=pl.BlockSpec((_BLK, 1), lambda i: (i, 0)),
        out_shape=jax.ShapeDtypeStruct((BATCH, 1), jnp.float32),
    )(u4, m14, m24, usel, m1sel, m2sel, w1u, w1m, b1r, w2r)


def kernel(user_ids, movie_ids_1, movie_ids_2, user_table, movie_table,
           W1, b1, W2, b2):
    uid = user_ids.astype(jnp.int32)
    m1id = movie_ids_1.astype(jnp.int32)
    m2id = movie_ids_2.astype(jnp.int32)
    uidx = (uid & QMASK).reshape(NW, NCH, CHUNK)
    m1idx = (m1id & QMASK).reshape(NW, NCH, CHUNK)
    m2idx = (m2id & QMASK).reshape(NW, NCH, CHUNK)
    utp, mtp = _tc_repack(user_table.T, movie_table.T)
    u4, m14, m24 = _sc_gather(utp, mtp, uidx, m1idx, m2idx)
    return _tc_mlp(u4, m14, m24,
                   (uid >> QSHIFT).reshape(BATCH, 1),
                   (m1id >> QSHIFT).reshape(BATCH, 1),
                   (m2id >> QSHIFT).reshape(BATCH, 1),
                   W1, b1, W2)


# speedup vs baseline: 4.1419x; 1.4408x over previous
import functools

import jax
import jax.numpy as jnp
from jax import lax
from jax.experimental import pallas as pl
from jax.experimental.pallas import tpu as pltpu
from jax.experimental.pallas import tpu_sc as plsc

BATCH = 16384
EMBED_DIM = 32
HIDDEN_DIM = 64
CHUNK = 128
PACK = 128 // EMBED_DIM

_info = plsc.get_sparse_core_info()
NC, NS = _info.num_cores, _info.num_subcores
NW = NC * NS
B_PER_W = BATCH // NW
NCH = B_PER_W // CHUNK

NROWS = 1000000
QSHIFT = 18
NSLICE = 1 << QSHIFT
QMASK = NSLICE - 1

RB = 2048
RGRID = NSLICE // RB
MAXB = (NROWS - 1) // RB


def _repack_body(eye_ref, u0, u1, u2, u3, m0, m1, m2, m3, uo_ref, mo_ref):
    dn = (((0,), (0,)), ((), ()))

    def pack(blocks):
        acc = None
        for p, r in enumerate(blocks):
            t = lax.dot_general(r[...], eye_ref[32 * p:32 * (p + 1), :],
                                dn, preferred_element_type=jnp.float32)
            acc = t if acc is None else acc + t
        return acc

    uo_ref[...] = pack((u0, u1, u2, u3))
    mo_ref[...] = pack((m0, m1, m2, m3))


def _tc_repack(tt_u, tt_m):
    out_t = jax.ShapeDtypeStruct((NSLICE, 128), jnp.float32)

    def spec(p):
        return pl.BlockSpec(
            (32, RB), lambda j, p=p: (0, jnp.minimum(128 * p + j, MAXB)))

    return pl.pallas_call(
        _repack_body,
        grid=(RGRID,),
        in_specs=[pl.BlockSpec((128, 128), lambda j: (0, 0))]
        + [spec(p) for p in range(4)] * 2,
        out_specs=[pl.BlockSpec((RB, 128), lambda j: (j, 0))] * 2,
        out_shape=[out_t, out_t],
    )(jnp.eye(128, dtype=jnp.float32), *([tt_u] * 4 + [tt_m] * 4))


def _sc_gather(ut, mt, uidx, m1idx, m2idx):
    mesh = plsc.VectorSubcoreMesh(core_axis_name="c", subcore_axis_name="s")
    out_t = jax.ShapeDtypeStruct((BATCH, 128), jnp.float32)

    @functools.partial(
        pl.kernel,
        mesh=mesh,
        out_type=[out_t, out_t, out_t],
        compiler_params=pltpu.CompilerParams(use_tc_tiling_on_sc=True),
        scratch_types=[
            pltpu.VMEM((NCH, CHUNK), jnp.int32),
            pltpu.VMEM((NCH, CHUNK), jnp.int32),
            pltpu.VMEM((NCH, CHUNK), jnp.int32),
        ] + [pltpu.VMEM((CHUNK, 128), jnp.float32) for _ in range(6)]
          + [pltpu.SemaphoreType.DMA for _ in range(6)],
    )
    def k(ut_hbm, mt_hbm, ui_hbm, m1i_hbm, m2i_hbm,
          u_out, m1_out, m2_out,
          ui_v, m1i_v, m2i_v, b0, b1_, b2_, b3, b4, b5,
          s0, s1, s2, s3, s4, s5):
        wid = lax.axis_index("s") * NC + lax.axis_index("c")
        base = wid * B_PER_W
        pltpu.sync_copy(ui_hbm.at[wid], ui_v)
        pltpu.sync_copy(m1i_hbm.at[wid], m1i_v)
        pltpu.sync_copy(m2i_hbm.at[wid], m2i_v)
        tabs = [(ut_hbm, ui_v, u_out), (mt_hbm, m1i_v, m1_out),
                (mt_hbm, m2i_v, m2_out)]
        bufs = [b0, b1_, b2_, b3, b4, b5]
        sems = [s0, s1, s2, s3, s4, s5]
        gd = {}
        for t in range(3):
            for s in range(2):
                tbl, idxv, _ = tabs[t]
                gd[(t, s)] = pltpu.async_copy(
                    tbl.at[idxv.at[s]], bufs[2 * t + s], sems[2 * t + s])
        od = {}
        for rnd in range(NCH // 2):
            for t in range(3):
                for s in range(2):
                    ch = 2 * rnd + s
                    tbl, idxv, out = tabs[t]
                    gd[(t, s)].wait()
                    od[(t, s)] = pltpu.async_copy(
                        bufs[2 * t + s],
                        out.at[pl.ds(base + ch * CHUNK, CHUNK)],
                        sems[2 * t + s])
            if rnd + 1 < NCH // 2:
                for t in range(3):
                    for s in range(2):
                        tbl, idxv, _ = tabs[t]
                        od[(t, s)].wait()
                        gd[(t, s)] = pltpu.async_copy(
                            tbl.at[idxv.at[2 * (rnd + 1) + s]],
                            bufs[2 * t + s], sems[2 * t + s])
        for t in range(3):
            for s in range(2):
                od[(t, s)].wait()

    return k(ut, mt, uidx, m1idx, m2idx)


_BLK = 2048


def _mlp_body(u_ref, m1_ref, m2_ref, us_ref, m1s_ref, m2s_ref,
              w1u_ref, w1m_ref, b1_ref, w2_ref, out_ref):
    def pick(x4, sel):
        r = jnp.where(sel == 0, x4[:, 0 * EMBED_DIM:1 * EMBED_DIM], 0.0)
        for kk in range(1, PACK):
            r = r + jnp.where(sel == kk,
                              x4[:, kk * EMBED_DIM:(kk + 1) * EMBED_DIM], 0.0)
        return r

    u = pick(u_ref[...], us_ref[...])
    m1 = pick(m1_ref[...], m1s_ref[...])
    m2 = pick(m2_ref[...], m2s_ref[...])
    U = jnp.dot(u, w1u_ref[...], preferred_element_type=jnp.float32)
    M1 = jnp.dot(m1, w1m_ref[...], preferred_element_type=jnp.float32)
    M2 = jnp.dot(m2, w1m_ref[...], preferred_element_type=jnp.float32)
    b1r = b1_ref[...]
    h1 = jnp.maximum(U + M1 + b1r, 0.0)
    h2 = jnp.maximum(U + M2 + b1r, 0.0)
    out_ref[...] = jnp.sum((h1 - h2) * w2_ref[...], axis=1, keepdims=True)


def _tc_mlp(u4, m14, m24, usel, m1sel, m2sel, W1, b1, W2):
    w1u = W1[:EMBED_DIM]
    w1m = W1[EMBED_DIM:]
    b1r = b1.reshape(1, HIDDEN_DIM)
    w2r = W2.reshape(1, HIDDEN_DIM)
    grid = (BATCH // _BLK,)
    return pl.pallas_call(
        _mlp_body,
        grid=grid,
        in_specs=[
            pl.BlockSpec((_BLK, 128), lambda i: (i, 0)),
            pl.BlockSpec((_BLK, 128), lambda i: (i, 0)),
            pl.BlockSpec((_BLK, 128), lambda i: (i, 0)),
            pl.BlockSpec((_BLK, 1), lambda i: (i, 0)),
            pl.BlockSpec((_BLK, 1), lambda i: (i, 0)),
            pl.BlockSpec((_BLK, 1), lambda i: (i, 0)),
            pl.BlockSpec((EMBED_DIM, HIDDEN_DIM), lambda i: (0, 0)),
            pl.BlockSpec((EMBED_DIM, HIDDEN_DIM), lambda i: (0, 0)),
            pl.BlockSpec((1, HIDDEN_DIM), lambda i: (0, 0)),
            pl.BlockSpec((1, HIDDEN_DIM), lambda i: (0, 0)),
        ],
        out_specs=pl.BlockSpec((_BLK, 1), lambda i: (i, 0)),
        out_shape=jax.ShapeDtypeStruct((BATCH, 1), jnp.float32),
    )(u4, m14, m24, usel, m1sel, m2sel, w1u, w1m, b1r, w2r)


def kernel(user_ids, movie_ids_1, movie_ids_2, user_table, movie_table,
           W1, b1, W2, b2):
    uid = user_ids.astype(jnp.int32)
    m1id = movie_ids_1.astype(jnp.int32)
    m2id = movie_ids_2.astype(jnp.int32)
    uidx = (uid & QMASK).reshape(NW, NCH, CHUNK)
    m1idx = (m1id & QMASK).reshape(NW, NCH, CHUNK)
    m2idx = (m2id & QMASK).reshape(NW, NCH, CHUNK)
    utp, mtp = _tc_repack(user_table.T, movie_table.T)
    u4, m14, m24 = _sc_gather(utp, mtp, uidx, m1idx, m2idx)
    return _tc_mlp(u4, m14, m24,
                   (uid >> QSHIFT).reshape(BATCH, 1),
                   (m1id >> QSHIFT).reshape(BATCH, 1),
                   (m2id >> QSHIFT).reshape(BATCH, 1),
                   W1, b1, W2)


# speedup vs baseline: 4.6916x; 1.1327x over previous
import functools

import jax
import jax.numpy as jnp
from jax import lax
from jax.experimental import pallas as pl
from jax.experimental.pallas import tpu as pltpu
from jax.experimental.pallas import tpu_sc as plsc

BATCH = 16384
EMBED_DIM = 32
HIDDEN_DIM = 64
CHUNK = 128
PACK = 128 // EMBED_DIM

_info = plsc.get_sparse_core_info()
NC, NS = _info.num_cores, _info.num_subcores
NW = NC * NS
B_PER_W = BATCH // NW
NCH = B_PER_W // CHUNK

NROWS = 1000000
QSHIFT = 18
NSLICE = 1 << QSHIFT
QMASK = NSLICE - 1

RB = 4096
RGRID = NSLICE // RB
MAXB = (NROWS - 1) // RB


def _repack_body(eye_ref, u0, u1, u2, u3, m0, m1, m2, m3, uo_ref, mo_ref):
    dn = (((0,), (0,)), ((), ()))

    def pack(blocks):
        acc = None
        for p, r in enumerate(blocks):
            t = lax.dot_general(r[...], eye_ref[32 * p:32 * (p + 1), :],
                                dn, preferred_element_type=jnp.float32)
            acc = t if acc is None else acc + t
        return acc

    uo_ref[...] = pack((u0, u1, u2, u3))
    mo_ref[...] = pack((m0, m1, m2, m3))


def _tc_repack(tt_u, tt_m):
    out_t = jax.ShapeDtypeStruct((NSLICE, 128), jnp.float32)

    def spec(p):
        return pl.BlockSpec(
            (32, RB), lambda j, p=p: (0, jnp.minimum(RGRID * p + j, MAXB)))

    return pl.pallas_call(
        _repack_body,
        grid=(RGRID,),
        in_specs=[pl.BlockSpec((128, 128), lambda j: (0, 0))]
        + [spec(p) for p in range(4)] * 2,
        out_specs=[pl.BlockSpec((RB, 128), lambda j: (j, 0))] * 2,
        out_shape=[out_t, out_t],
    )(jnp.eye(128, dtype=jnp.float32), *([tt_u] * 4 + [tt_m] * 4))


def _sc_gather(ut, mt, uidx, m1idx, m2idx):
    mesh = plsc.VectorSubcoreMesh(core_axis_name="c", subcore_axis_name="s")
    out_t = jax.ShapeDtypeStruct((BATCH, 128), jnp.float32)

    @functools.partial(
        pl.kernel,
        mesh=mesh,
        out_type=[out_t, out_t, out_t],
        compiler_params=pltpu.CompilerParams(use_tc_tiling_on_sc=True),
        scratch_types=[
            pltpu.VMEM((NCH, CHUNK), jnp.int32),
            pltpu.VMEM((NCH, CHUNK), jnp.int32),
            pltpu.VMEM((NCH, CHUNK), jnp.int32),
        ] + [pltpu.VMEM((CHUNK, 128), jnp.float32) for _ in range(6)]
          + [pltpu.SemaphoreType.DMA for _ in range(6)],
    )
    def k(ut_hbm, mt_hbm, ui_hbm, m1i_hbm, m2i_hbm,
          u_out, m1_out, m2_out,
          ui_v, m1i_v, m2i_v, b0, b1_, b2_, b3, b4, b5,
          s0, s1, s2, s3, s4, s5):
        wid = lax.axis_index("s") * NC + lax.axis_index("c")
        base = wid * B_PER_W
        pltpu.sync_copy(ui_hbm.at[wid], ui_v)
        pltpu.sync_copy(m1i_hbm.at[wid], m1i_v)
        pltpu.sync_copy(m2i_hbm.at[wid], m2i_v)
        tabs = [(ut_hbm, ui_v, u_out), (mt_hbm, m1i_v, m1_out),
                (mt_hbm, m2i_v, m2_out)]
        bufs = [b0, b1_, b2_, b3, b4, b5]
        sems = [s0, s1, s2, s3, s4, s5]
        gd = {}
        for t in range(3):
            for s in range(2):
                tbl, idxv, _ = tabs[t]
                gd[(t, s)] = pltpu.async_copy(
                    tbl.at[idxv.at[s]], bufs[2 * t + s], sems[2 * t + s])
        od = {}
        for rnd in range(NCH // 2):
            for t in range(3):
                for s in range(2):
                    ch = 2 * rnd + s
                    tbl, idxv, out = tabs[t]
                    gd[(t, s)].wait()
                    od[(t, s)] = pltpu.async_copy(
                        bufs[2 * t + s],
                        out.at[pl.ds(base + ch * CHUNK, CHUNK)],
                        sems[2 * t + s])
            if rnd + 1 < NCH // 2:
                for t in range(3):
                    for s in range(2):
                        tbl, idxv, _ = tabs[t]
                        od[(t, s)].wait()
                        gd[(t, s)] = pltpu.async_copy(
                            tbl.at[idxv.at[2 * (rnd + 1) + s]],
                            bufs[2 * t + s], sems[2 * t + s])
        for t in range(3):
            for s in range(2):
                od[(t, s)].wait()

    return k(ut, mt, uidx, m1idx, m2idx)


_BLK = 2048


def _mlp_body(u_ref, m1_ref, m2_ref, us_ref, m1s_ref, m2s_ref,
              w1u_ref, w1m_ref, b1_ref, w2_ref, out_ref):
    lane_grp = lax.broadcasted_iota(jnp.int32, (1, 128), 1) // EMBED_DIM

    def term(x_ref, s_ref, w_ref):
        m = (lane_grp == s_ref[...]).astype(jnp.float32)
        return jnp.dot(x_ref[...] * m, w_ref[...],
                       preferred_element_type=jnp.float32)

    U = term(u_ref, us_ref, w1u_ref)
    M1 = term(m1_ref, m1s_ref, w1m_ref)
    M2 = term(m2_ref, m2s_ref, w1m_ref)
    b1r = b1_ref[...]
    h1 = jnp.maximum(U + M1 + b1r, 0.0)
    h2 = jnp.maximum(U + M2 + b1r, 0.0)
    out_ref[...] = jnp.sum((h1 - h2) * w2_ref[...], axis=1, keepdims=True)


def _tc_mlp(u4, m14, m24, usel, m1sel, m2sel, W1, b1, W2):
    w1u = jnp.tile(W1[:EMBED_DIM], (PACK, 1))
    w1m = jnp.tile(W1[EMBED_DIM:], (PACK, 1))
    b1r = b1.reshape(1, HIDDEN_DIM)
    w2r = W2.reshape(1, HIDDEN_DIM)
    grid = (BATCH // _BLK,)
    return pl.pallas_call(
        _mlp_body,
        grid=grid,
        in_specs=[
            pl.BlockSpec((_BLK, 128), lambda i: (i, 0)),
            pl.BlockSpec((_BLK, 128), lambda i: (i, 0)),
            pl.BlockSpec((_BLK, 128), lambda i: (i, 0)),
            pl.BlockSpec((_BLK, 1), lambda i: (i, 0)),
            pl.BlockSpec((_BLK, 1), lambda i: (i, 0)),
            pl.BlockSpec((_BLK, 1), lambda i: (i, 0)),
            pl.BlockSpec((PACK * EMBED_DIM, HIDDEN_DIM), lambda i: (0, 0)),
            pl.BlockSpec((PACK * EMBED_DIM, HIDDEN_DIM), lambda i: (0, 0)),
            pl.BlockSpec((1, HIDDEN_DIM), lambda i: (0, 0)),
            pl.BlockSpec((1, HIDDEN_DIM), lambda i: (0, 0)),
        ],
        out_specs=pl.BlockSpec((_BLK, 1), lambda i: (i, 0)),
        out_shape=jax.ShapeDtypeStruct((BATCH, 1), jnp.float32),
    )(u4, m14, m24, usel, m1sel, m2sel, w1u, w1m, b1r, w2r)


def kernel(user_ids, movie_ids_1, movie_ids_2, user_table, movie_table,
           W1, b1, W2, b2):
    uid = user_ids.astype(jnp.int32)
    m1id = movie_ids_1.astype(jnp.int32)
    m2id = movie_ids_2.astype(jnp.int32)
    uidx = (uid & QMASK).reshape(NW, NCH, CHUNK)
    m1idx = (m1id & QMASK).reshape(NW, NCH, CHUNK)
    m2idx = (m2id & QMASK).reshape(NW, NCH, CHUNK)
    utp, mtp = _tc_repack(user_table.T, movie_table.T)
    u4, m14, m24 = _sc_gather(utp, mtp, uidx, m1idx, m2idx)
    return _tc_mlp(u4, m14, m24,
                   (uid >> QSHIFT).reshape(BATCH, 1),
                   (m1id >> QSHIFT).reshape(BATCH, 1),
                   (m2id >> QSHIFT).reshape(BATCH, 1),
                   W1, b1, W2)


# speedup vs baseline: 4.7959x; 1.0222x over previous
import functools

import jax
import jax.numpy as jnp
from jax import lax
from jax.experimental import pallas as pl
from jax.experimental.pallas import tpu as pltpu
from jax.experimental.pallas import tpu_sc as plsc

BATCH = 16384
EMBED_DIM = 32
HIDDEN_DIM = 64
CHUNK = 128
PACK = 128 // EMBED_DIM

_info = plsc.get_sparse_core_info()
NC, NS = _info.num_cores, _info.num_subcores
NW = NC * NS
B_PER_W = BATCH // NW
NCH = B_PER_W // CHUNK

NROWS = 1000000
QSHIFT = 18
NSLICE = 1 << QSHIFT
QMASK = NSLICE - 1

RB = 8192
RGRID = NSLICE // RB
MAXB = (NROWS - 1) // RB


def _repack_body(eye_ref, u0, u1, u2, u3, m0, m1, m2, m3, uo_ref, mo_ref):
    dn = (((0,), (0,)), ((), ()))

    def pack(blocks):
        acc = None
        for p, r in enumerate(blocks):
            t = lax.dot_general(r[...], eye_ref[32 * p:32 * (p + 1), :],
                                dn, preferred_element_type=jnp.float32)
            acc = t if acc is None else acc + t
        return acc

    uo_ref[...] = pack((u0, u1, u2, u3))
    mo_ref[...] = pack((m0, m1, m2, m3))


def _tc_repack(tt_u, tt_m):
    out_t = jax.ShapeDtypeStruct((NSLICE, 128), jnp.float32)

    def spec(p):
        return pl.BlockSpec(
            (32, RB), lambda j, p=p: (0, jnp.minimum(RGRID * p + j, MAXB)))

    return pl.pallas_call(
        _repack_body,
        grid=(RGRID,),
        in_specs=[pl.BlockSpec((128, 128), lambda j: (0, 0))]
        + [spec(p) for p in range(4)] * 2,
        out_specs=[pl.BlockSpec((RB, 128), lambda j: (j, 0))] * 2,
        out_shape=[out_t, out_t],
    )(jnp.eye(128, dtype=jnp.float32), *([tt_u] * 4 + [tt_m] * 4))


def _sc_gather(ut, mt, uidx, m1idx, m2idx):
    mesh = plsc.VectorSubcoreMesh(core_axis_name="c", subcore_axis_name="s")
    out_t = jax.ShapeDtypeStruct((BATCH, 128), jnp.float32)

    @functools.partial(
        pl.kernel,
        mesh=mesh,
        out_type=[out_t, out_t, out_t],
        compiler_params=pltpu.CompilerParams(use_tc_tiling_on_sc=True),
        scratch_types=[
            pltpu.VMEM((NCH, CHUNK), jnp.int32),
            pltpu.VMEM((NCH, CHUNK), jnp.int32),
            pltpu.VMEM((NCH, CHUNK), jnp.int32),
        ] + [pltpu.VMEM((CHUNK, 128), jnp.float32) for _ in range(6)]
          + [pltpu.SemaphoreType.DMA for _ in range(6)],
    )
    def k(ut_hbm, mt_hbm, ui_hbm, m1i_hbm, m2i_hbm,
          u_out, m1_out, m2_out,
          ui_v, m1i_v, m2i_v, b0, b1_, b2_, b3, b4, b5,
          s0, s1, s2, s3, s4, s5):
        wid = lax.axis_index("s") * NC + lax.axis_index("c")
        base = wid * B_PER_W
        pltpu.sync_copy(ui_hbm.at[wid], ui_v)
        pltpu.sync_copy(m1i_hbm.at[wid], m1i_v)
        pltpu.sync_copy(m2i_hbm.at[wid], m2i_v)
        tabs = [(ut_hbm, ui_v, u_out), (mt_hbm, m1i_v, m1_out),
                (mt_hbm, m2i_v, m2_out)]
        bufs = [b0, b1_, b2_, b3, b4, b5]
        sems = [s0, s1, s2, s3, s4, s5]
        gd = {}
        for t in range(3):
            for s in range(2):
                tbl, idxv, _ = tabs[t]
                gd[(t, s)] = pltpu.async_copy(
                    tbl.at[idxv.at[s]], bufs[2 * t + s], sems[2 * t + s])
        od = {}
        for rnd in range(NCH // 2):
            for t in range(3):
                for s in range(2):
                    ch = 2 * rnd + s
                    tbl, idxv, out = tabs[t]
                    gd[(t, s)].wait()
                    od[(t, s)] = pltpu.async_copy(
                        bufs[2 * t + s],
                        out.at[pl.ds(base + ch * CHUNK, CHUNK)],
                        sems[2 * t + s])
            if rnd + 1 < NCH // 2:
                for t in range(3):
                    for s in range(2):
                        tbl, idxv, _ = tabs[t]
                        od[(t, s)].wait()
                        gd[(t, s)] = pltpu.async_copy(
                            tbl.at[idxv.at[2 * (rnd + 1) + s]],
                            bufs[2 * t + s], sems[2 * t + s])
        for t in range(3):
            for s in range(2):
                od[(t, s)].wait()

    return k(ut, mt, uidx, m1idx, m2idx)


_BLK = 2048


def _mlp_body(u_ref, m1_ref, m2_ref, us_ref, m1s_ref, m2s_ref,
              w1u_ref, w1m_ref, b1_ref, w2_ref, out_ref):
    lane_grp = lax.broadcasted_iota(jnp.int32, (1, 128), 1) // EMBED_DIM

    def term(x_ref, s_ref, w_ref):
        m = (lane_grp == s_ref[...]).astype(jnp.float32)
        return jnp.dot(x_ref[...] * m, w_ref[...],
                       preferred_element_type=jnp.float32)

    U = term(u_ref, us_ref, w1u_ref)
    M1 = term(m1_ref, m1s_ref, w1m_ref)
    M2 = term(m2_ref, m2s_ref, w1m_ref)
    b1r = b1_ref[...]
    h1 = jnp.maximum(U + M1 + b1r, 0.0)
    h2 = jnp.maximum(U + M2 + b1r, 0.0)
    out_ref[...] = jnp.sum((h1 - h2) * w2_ref[...], axis=1, keepdims=True)


def _tc_mlp(u4, m14, m24, usel, m1sel, m2sel, W1, b1, W2):
    w1u = jnp.tile(W1[:EMBED_DIM], (PACK, 1))
    w1m = jnp.tile(W1[EMBED_DIM:], (PACK, 1))
    b1r = b1.reshape(1, HIDDEN_DIM)
    w2r = W2.reshape(1, HIDDEN_DIM)
    grid = (BATCH // _BLK,)
    return pl.pallas_call(
        _mlp_body,
        grid=grid,
        in_specs=[
            pl.BlockSpec((_BLK, 128), lambda i: (i, 0)),
            pl.BlockSpec((_BLK, 128), lambda i: (i, 0)),
            pl.BlockSpec((_BLK, 128), lambda i: (i, 0)),
            pl.BlockSpec((_BLK, 1), lambda i: (i, 0)),
            pl.BlockSpec((_BLK, 1), lambda i: (i, 0)),
            pl.BlockSpec((_BLK, 1), lambda i: (i, 0)),
            pl.BlockSpec((PACK * EMBED_DIM, HIDDEN_DIM), lambda i: (0, 0)),
            pl.BlockSpec((PACK * EMBED_DIM, HIDDEN_DIM), lambda i: (0, 0)),
            pl.BlockSpec((1, HIDDEN_DIM), lambda i: (0, 0)),
            pl.BlockSpec((1, HIDDEN_DIM), lambda i: (0, 0)),
        ],
        out_specs=pl.BlockSpec((_BLK, 1), lambda i: (i, 0)),
        out_shape=jax.ShapeDtypeStruct((BATCH, 1), jnp.float32),
    )(u4, m14, m24, usel, m1sel, m2sel, w1u, w1m, b1r, w2r)


def kernel(user_ids, movie_ids_1, movie_ids_2, user_table, movie_table,
           W1, b1, W2, b2):
    uid = user_ids.astype(jnp.int32)
    m1id = movie_ids_1.astype(jnp.int32)
    m2id = movie_ids_2.astype(jnp.int32)
    uidx = (uid & QMASK).reshape(NW, NCH, CHUNK)
    m1idx = (m1id & QMASK).reshape(NW, NCH, CHUNK)
    m2idx = (m2id & QMASK).reshape(NW, NCH, CHUNK)
    utp, mtp = _tc_repack(user_table.T, movie_table.T)
    u4, m14, m24 = _sc_gather(utp, mtp, uidx, m1idx, m2idx)
    return _tc_mlp(u4, m14, m24,
                   (uid >> QSHIFT).reshape(BATCH, 1),
                   (m1id >> QSHIFT).reshape(BATCH, 1),
                   (m2id >> QSHIFT).reshape(BATCH, 1),
                   W1, b1, W2)


# speedup vs baseline: 7.3920x; 1.5413x over previous
import functools

import jax
import jax.numpy as jnp
from jax import lax
from jax.experimental import pallas as pl
from jax.experimental.pallas import tpu as pltpu
from jax.experimental.pallas import tpu_sc as plsc

BATCH = 16384
EMBED_DIM = 32
HIDDEN_DIM = 64
CHUNK = 128
PACK = 128 // EMBED_DIM

_info = plsc.get_sparse_core_info()
NC, NS = _info.num_cores, _info.num_subcores
NW = NC * NS
B_PER_W = BATCH // NW
NCH = B_PER_W // CHUNK

NROWS = 1000000
QSHIFT = 18
NSLICE = 1 << QSHIFT
QMASK = NSLICE - 1

RB = 8192
RGRID = NSLICE // RB
MAXB = (NROWS - 1) // RB


def _repack_body(eye_ref, u0, u1, u2, u3, m0, m1, m2, m3, uo_ref, mo_ref):
    dn = (((0,), (0,)), ((), ()))

    def pack(blocks):
        x = jnp.concatenate([r[...] for r in blocks], axis=0)
        return lax.dot_general(x, eye_ref[...], dn,
                               preferred_element_type=jnp.float32)

    uo_ref[...] = pack((u0, u1, u2, u3))
    mo_ref[...] = pack((m0, m1, m2, m3))


def _tc_repack(tt_u, tt_m):
    out_t = jax.ShapeDtypeStruct((NSLICE, 128), jnp.float32)

    def spec(p):
        return pl.BlockSpec(
            (32, RB), lambda j, p=p: (0, jnp.minimum(RGRID * p + j, MAXB)))

    return pl.pallas_call(
        _repack_body,
        grid=(RGRID,),
        in_specs=[pl.BlockSpec((128, 128), lambda j: (0, 0))]
        + [spec(p) for p in range(4)] * 2,
        out_specs=[pl.BlockSpec((RB, 128), lambda j: (j, 0))] * 2,
        out_shape=[out_t, out_t],
    )(jnp.eye(128, dtype=jnp.float32), *([tt_u] * 4 + [tt_m] * 4))


def _sc_gather(ut, mt, uidx, m1idx, m2idx):
    mesh = plsc.VectorSubcoreMesh(core_axis_name="c", subcore_axis_name="s")
    out_t = jax.ShapeDtypeStruct((BATCH, 128), jnp.float32)

    @functools.partial(
        pl.kernel,
        mesh=mesh,
        out_type=[out_t, out_t, out_t],
        compiler_params=pltpu.CompilerParams(use_tc_tiling_on_sc=True),
        scratch_types=[
            pltpu.VMEM((NCH, CHUNK), jnp.int32),
            pltpu.VMEM((NCH, CHUNK), jnp.int32),
            pltpu.VMEM((NCH, CHUNK), jnp.int32),
        ] + [pltpu.VMEM((CHUNK, 128), jnp.float32) for _ in range(6)]
          + [pltpu.SemaphoreType.DMA for _ in range(6)],
    )
    def k(ut_hbm, mt_hbm, ui_hbm, m1i_hbm, m2i_hbm,
          u_out, m1_out, m2_out,
          ui_v, m1i_v, m2i_v, b0, b1_, b2_, b3, b4, b5,
          s0, s1, s2, s3, s4, s5):
        wid = lax.axis_index("s") * NC + lax.axis_index("c")
        base = wid * B_PER_W
        pltpu.sync_copy(ui_hbm.at[wid], ui_v)
        pltpu.sync_copy(m1i_hbm.at[wid], m1i_v)
        pltpu.sync_copy(m2i_hbm.at[wid], m2i_v)
        tabs = [(ut_hbm, ui_v, u_out), (mt_hbm, m1i_v, m1_out),
                (mt_hbm, m2i_v, m2_out)]
        bufs = [b0, b1_, b2_, b3, b4, b5]
        sems = [s0, s1, s2, s3, s4, s5]
        gd = {}
        for t in range(3):
            for s in range(2):
                tbl, idxv, _ = tabs[t]
                gd[(t, s)] = pltpu.async_copy(
                    tbl.at[idxv.at[s]], bufs[2 * t + s], sems[2 * t + s])
        od = {}
        for rnd in range(NCH // 2):
            for t in range(3):
                for s in range(2):
                    ch = 2 * rnd + s
                    tbl, idxv, out = tabs[t]
                    gd[(t, s)].wait()
                    od[(t, s)] = pltpu.async_copy(
                        bufs[2 * t + s],
                        out.at[pl.ds(base + ch * CHUNK, CHUNK)],
                        sems[2 * t + s])
            if rnd + 1 < NCH // 2:
                for t in range(3):
                    for s in range(2):
                        tbl, idxv, _ = tabs[t]
                        od[(t, s)].wait()
                        gd[(t, s)] = pltpu.async_copy(
                            tbl.at[idxv.at[2 * (rnd + 1) + s]],
                            bufs[2 * t + s], sems[2 * t + s])
        for t in range(3):
            for s in range(2):
                od[(t, s)].wait()

    return k(ut, mt, uidx, m1idx, m2idx)


_BLK = 2048


def _mlp_body(u_ref, m1_ref, m2_ref, us_ref, m1s_ref, m2s_ref,
              w1u_ref, w1m_ref, b1_ref, w2_ref, out_ref):
    lane_grp = lax.broadcasted_iota(jnp.int32, (1, 128), 1) // EMBED_DIM

    def term(x_ref, s_ref, w_ref):
        m = (lane_grp == s_ref[...]).astype(jnp.float32)
        return jnp.dot(x_ref[...] * m, w_ref[...],
                       preferred_element_type=jnp.float32)

    U = term(u_ref, us_ref, w1u_ref)
    M1 = term(m1_ref, m1s_ref, w1m_ref)
    M2 = term(m2_ref, m2s_ref, w1m_ref)
    b1r = b1_ref[...]
    h1 = jnp.maximum(U + M1 + b1r, 0.0)
    h2 = jnp.maximum(U + M2 + b1r, 0.0)
    out_ref[...] = jnp.sum((h1 - h2) * w2_ref[...], axis=1, keepdims=True)


def _tc_mlp(u4, m14, m24, usel, m1sel, m2sel, W1, b1, W2):
    w1u = jnp.tile(W1[:EMBED_DIM], (PACK, 1))
    w1m = jnp.tile(W1[EMBED_DIM:], (PACK, 1))
    b1r = b1.reshape(1, HIDDEN_DIM)
    w2r = W2.reshape(1, HIDDEN_DIM)
    grid = (BATCH // _BLK,)
    return pl.pallas_call(
        _mlp_body,
        grid=grid,
        in_specs=[
            pl.BlockSpec((_BLK, 128), lambda i: (i, 0)),
            pl.BlockSpec((_BLK, 128), lambda i: (i, 0)),
            pl.BlockSpec((_BLK, 128), lambda i: (i, 0)),
            pl.BlockSpec((_BLK, 1), lambda i: (i, 0)),
            pl.BlockSpec((_BLK, 1), lambda i: (i, 0)),
            pl.BlockSpec((_BLK, 1), lambda i: (i, 0)),
            pl.BlockSpec((PACK * EMBED_DIM, HIDDEN_DIM), lambda i: (0, 0)),
            pl.BlockSpec((PACK * EMBED_DIM, HIDDEN_DIM), lambda i: (0, 0)),
            pl.BlockSpec((1, HIDDEN_DIM), lambda i: (0, 0)),
            pl.BlockSpec((1, HIDDEN_DIM), lambda i: (0, 0)),
        ],
        out_specs=pl.BlockSpec((_BLK, 1), lambda i: (i, 0)),
        out_shape=jax.ShapeDtypeStruct((BATCH, 1), jnp.float32),
    )(u4, m14, m24, usel, m1sel, m2sel, w1u, w1m, b1r, w2r)


def kernel(user_ids, movie_ids_1, movie_ids_2, user_table, movie_table,
           W1, b1, W2, b2):
    uid = user_ids.astype(jnp.int32)
    m1id = movie_ids_1.astype(jnp.int32)
    m2id = movie_ids_2.astype(jnp.int32)
    uidx = (uid & QMASK).reshape(NW, NCH, CHUNK)
    m1idx = (m1id & QMASK).reshape(NW, NCH, CHUNK)
    m2idx = (m2id & QMASK).reshape(NW, NCH, CHUNK)
    utp, mtp = _tc_repack(user_table.T, movie_table.T)
    u4, m14, m24 = _sc_gather(utp, mtp, uidx, m1idx, m2idx)
    return _tc_mlp(u4, m14, m24,
                   (uid >> QSHIFT).reshape(BATCH, 1),
                   (m1id >> QSHIFT).reshape(BATCH, 1),
                   (m2id >> QSHIFT).reshape(BATCH, 1),
                   W1, b1, W2)
